# baseline (speedup 1.0000x reference)
import functools

import jax
import jax.numpy as jnp
from jax import lax
from jax.experimental import pallas as pl
from jax.experimental.pallas import tpu as pltpu
from jax.experimental.pallas import tpu_sc as plsc
from jax.experimental import layout as jex_layout

_B = 4096
_DIN = 13
_D = 32
_DP = 33
_NL = 22
_VL = 100000
_NS = 4
_VS = 1000
_L = 20
_NW = 32
_NROW = _NL * _D
_RPW = _NROW // _NW
_HC = _VL // 2
_BAGS_PW = _B // 8

_sc_mesh = plsc.VectorSubcoreMesh(core_axis_name="c", subcore_axis_name="s")


_NLH = _NL // 2
_NROWH = _NLH * _D
_RPWH = _NROWH // _NW


def _make_large_gather():
    @functools.partial(
        pl.kernel,
        out_type=jax.ShapeDtypeStruct((_NROWH, _B), jnp.float32),
        mesh=_sc_mesh,
        compiler_params=pltpu.CompilerParams(
            needs_layout_passes=False, use_tc_tiling_on_sc=False),
        scratch_types=[
            pltpu.VMEM((_VL,), jnp.float32),
            pltpu.VMEM((_B,), jnp.float32),
            pltpu.VMEM((2 * _B,), jnp.int32),
            pltpu.SemaphoreType.DMA,
        ],
    )
    def lg(liT_hbm, tabT_hbm, lout_hbm, rowb, outb, ia, gsem):
        wid = lax.axis_index("s") * 2 + lax.axis_index("c")
        r0 = wid * _RPWH
        j0 = r0 // _D
        j1 = jnp.minimum(j0 + 1, _NLH - 1)
        pltpu.sync_copy(liT_hbm.at[j0], ia.at[pl.ds(0, _B)])
        pltpu.sync_copy(liT_hbm.at[j1], ia.at[pl.ds(_B, _B)])

        def row_body(ri, carry):
            r = r0 + ri
            vb = (r // _D - j0) * _B
            h0 = pltpu.async_copy(tabT_hbm.at[r, pl.ds(0, _HC)],
                                  rowb.at[pl.ds(0, _HC)], gsem)
            h1 = pltpu.async_copy(tabT_hbm.at[r, pl.ds(_HC, _HC)],
                                  rowb.at[pl.ds(_HC, _HC)], gsem)
            h0.wait()
            h1.wait()

            def grp(g, c):
                v = ia[pl.ds(vb + g * 16, 16)]
                outb[pl.ds(g * 16, 16)] = plsc.load_gather(rowb, [v])
                return c

            lax.fori_loop(0, _B // 16, grp, 0)
            pltpu.sync_copy(outb, lout_hbm.at[r])
            return carry

        lax.fori_loop(0, _RPWH, row_body, 0)

    return lg


_large_gather0 = _make_large_gather()
_large_gather1 = _make_large_gather()


@functools.partial(
    pl.kernel,
    out_type=jax.ShapeDtypeStruct((_NW, _D, _BAGS_PW), jnp.float32),
    mesh=_sc_mesh,
    compiler_params=pltpu.CompilerParams(needs_layout_passes=False),
    scratch_types=[
        pltpu.VMEM((_VS * _DP,), jnp.float32),
        pltpu.VMEM((_L, _BAGS_PW), jnp.int32),
        pltpu.VMEM((_D, _BAGS_PW), jnp.float32),
    ],
)
def _small_bagsum(sidx_hbm, stab_hbm, sout_hbm, tab_v, idx_v, out_v):
    wid = lax.axis_index("s") * 2 + lax.axis_index("c")
    feat = wid // 8
    pltpu.sync_copy(stab_hbm.at[pl.ds(feat * (_VS * _DP), _VS * _DP)], tab_v)
    pltpu.sync_copy(sidx_hbm.at[wid], idx_v)

    def bag_body(g, carry):
        for half in range(2):
            accs = [jnp.zeros((16,), jnp.float32) for _ in range(16)]
            for l in range(_L):
                ids = idx_v[l, pl.ds(g * 16, 16)]
                w0 = ids * _DP + (half * 16)
                for c in range(16):
                    accs[c] = accs[c] + plsc.load_gather(tab_v, [w0 + c])
            for c in range(16):
                out_v[half * 16 + c, pl.ds(g * 16, 16)] = accs[c]
        return carry

    lax.fori_loop(0, _BAGS_PW // 16, bag_body, 0)
    pltpu.sync_copy(out_v, sout_hbm.at[wid])


_BB = 512


def _tc_body(dense_ref, small_ref, large_ref,
             bw0_ref, bb0_ref, bw1_ref, bb1_ref, bw2_ref, bb2_ref,
             U_ref, V_ref, db_ref,
             tw0_ref, tb0_ref, tw1_ref, tb1_ref, tw2_ref, tb2_ref,
             tw3_ref, tb3_ref, out_ref):
    f32 = jnp.float32
    h = dense_ref[...]
    h = jnp.maximum(jnp.dot(h, bw0_ref[...], preferred_element_type=f32)
                    + bb0_ref[...], 0.0)
    h = jnp.maximum(jnp.dot(h, bw1_ref[...], preferred_element_type=f32)
                    + bb1_ref[...], 0.0)
    h = jnp.maximum(jnp.dot(h, bw2_ref[...], preferred_element_type=f32)
                    + bb2_ref[...], 0.0)
    x0 = jnp.concatenate([h, small_ref[...], large_ref[...]], axis=1)
    xl = x0
    for i in range(3):
        t = jnp.dot(xl, U_ref[i], preferred_element_type=f32)
        t = jnp.dot(t, V_ref[i], preferred_element_type=f32) + db_ref[i]
        xl = x0 * t + xl
    h = jnp.maximum(jnp.dot(xl, tw0_ref[...], preferred_element_type=f32)
                    + tb0_ref[...], 0.0)
    h = jnp.maximum(jnp.dot(h, tw1_ref[...], preferred_element_type=f32)
                    + tb1_ref[...], 0.0)
    h = jnp.maximum(jnp.dot(h, tw2_ref[...], preferred_element_type=f32)
                    + tb2_ref[...], 0.0)
    out_ref[...] = jax.nn.sigmoid(
        jnp.dot(h, tw3_ref[...], preferred_element_type=f32) + tb3_ref[...])


def kernel(dense_input, large_emb_inputs, small_emb_inputs, large_tables,
           small_tables, bw0, bb0, bw1, bb1, bw2, bb2, dcn_U, dcn_V, dcn_b,
           tw0, tb0, tw1, tb1, tw2, tb2, tw3, tb3):
    f32 = jnp.float32
    si = small_emb_inputs.astype(jnp.int32)
    sidx = si.reshape(8, _BAGS_PW, _NS, _L).transpose(2, 0, 3, 1)
    sidx = sidx.reshape(_NW, _L, _BAGS_PW)
    stab = jnp.pad(small_tables.reshape(_NS * _VS, _D), ((0, 0), (0, 1)))
    sout = _small_bagsum(sidx, stab.reshape(_NS * _VS * _DP))
    small_cat = sout.reshape(_NS, 8, _D, _BAGS_PW).transpose(1, 3, 0, 2)
    small_cat = small_cat.reshape(_B, _NS * _D)

    li = large_emb_inputs.astype(jnp.int32).T
    tabT0 = jnp.transpose(large_tables[:_NLH], (0, 2, 1)).reshape(_NROWH, _VL)
    tabT1 = jnp.transpose(large_tables[_NLH:], (0, 2, 1)).reshape(_NROWH, _VL)
    lout0 = _large_gather0(li[:_NLH], tabT0)
    lout1 = _large_gather1(li[_NLH:], tabT1)
    large_cat = jnp.concatenate([lout0, lout1], axis=0).T

    row2 = lambda v: v.reshape(1, -1)
    full = lambda *shape: pl.BlockSpec(shape, lambda i: (0,) * len(shape))
    batched = lambda w: pl.BlockSpec((_BB, w), lambda i: (i, 0))

    out = pl.pallas_call(
        _tc_body,
        grid=(_B // _BB,),
        in_specs=[
            batched(_DIN), batched(_NS * _D), batched(_NL * _D),
            full(_DIN, 512), full(1, 512), full(512, 256), full(1, 256),
            full(256, 32), full(1, 32),
            full(3, 864, 128), full(3, 128, 864), full(3, 864),
            full(864, 1024), full(1, 1024), full(1024, 512), full(1, 512),
            full(512, 256), full(1, 256), full(256, 1), full(1, 1),
        ],
        out_specs=pl.BlockSpec((_BB, 1), lambda i: (i, 0)),
        out_shape=jax.ShapeDtypeStruct((_B, 1), f32),
    )(dense_input, small_cat, large_cat,
      bw0, row2(bb0), bw1, row2(bb1), bw2, row2(bb2),
      dcn_U, dcn_V, dcn_b,
      tw0, row2(tb0), tw1, row2(tb1), tw2, row2(tb2),
      tw3, row2(tb3))
    return out


# speedup vs baseline: 1.1991x; 1.1991x over previous
import functools

import jax
import jax.numpy as jnp
from jax import lax
from jax.experimental import pallas as pl
from jax.experimental.pallas import tpu as pltpu
from jax.experimental.pallas import tpu_sc as plsc
from jax.experimental import layout as jex_layout

_B = 4096
_DIN = 13
_D = 32
_DP = 33
_NL = 22
_VL = 100000
_NS = 4
_VS = 1000
_L = 20
_NW = 32
_NROW = _NL * _D
_RPW = _NROW // _NW
_HC = _VL // 2
_BAGS_PW = _B // 8

_sc_mesh = plsc.VectorSubcoreMesh(core_axis_name="c", subcore_axis_name="s")


@functools.partial(
    pl.kernel,
    out_type=jax.ShapeDtypeStruct((_NROW, _B), jnp.float32),
    mesh=_sc_mesh,
    compiler_params=pltpu.CompilerParams(
        needs_layout_passes=False, use_tc_tiling_on_sc=False),
    scratch_types=[
        pltpu.VMEM((_VL,), jnp.float32),
        pltpu.VMEM((_B,), jnp.float32),
        pltpu.VMEM((2 * _B,), jnp.int32),
        pltpu.SemaphoreType.DMA,
    ],
)
def _large_gather(liT_hbm, tabT_hbm, lout_hbm, rowb, outb, ia, gsem):
    wid = lax.axis_index("s") * 2 + lax.axis_index("c")
    r0 = wid * _RPW
    j0 = r0 // _D
    j1 = jnp.minimum(j0 + 1, _NL - 1)
    pltpu.sync_copy(liT_hbm.at[j0], ia.at[pl.ds(0, _B)])
    pltpu.sync_copy(liT_hbm.at[j1], ia.at[pl.ds(_B, _B)])

    def row_body(ri, carry):
        r = r0 + ri
        vb = (r // _D - j0) * _B
        h0 = pltpu.async_copy(tabT_hbm.at[r, pl.ds(0, _HC)],
                              rowb.at[pl.ds(0, _HC)], gsem)
        h1 = pltpu.async_copy(tabT_hbm.at[r, pl.ds(_HC, _HC)],
                              rowb.at[pl.ds(_HC, _HC)], gsem)
        h0.wait()
        h1.wait()

        def grp(g, c):
            v = ia[pl.ds(vb + g * 16, 16)]
            outb[pl.ds(g * 16, 16)] = plsc.load_gather(rowb, [v])
            return c

        lax.fori_loop(0, _B // 16, grp, 0)
        pltpu.sync_copy(outb, lout_hbm.at[r])
        return carry

    lax.fori_loop(0, _RPW, row_body, 0)


@functools.partial(
    pl.kernel,
    out_type=jax.ShapeDtypeStruct((_NW, _D, _BAGS_PW), jnp.float32),
    mesh=_sc_mesh,
    compiler_params=pltpu.CompilerParams(needs_layout_passes=False),
    scratch_types=[
        pltpu.VMEM((_VS * _DP,), jnp.float32),
        pltpu.VMEM((_L, _BAGS_PW), jnp.int32),
        pltpu.VMEM((_D, _BAGS_PW), jnp.float32),
    ],
)
def _small_bagsum(sidx_hbm, stab_hbm, sout_hbm, tab_v, idx_v, out_v):
    wid = lax.axis_index("s") * 2 + lax.axis_index("c")
    feat = wid // 8
    pltpu.sync_copy(stab_hbm.at[pl.ds(feat * (_VS * _DP), _VS * _DP)], tab_v)
    pltpu.sync_copy(sidx_hbm.at[wid], idx_v)

    def bag_body(g, carry):
        for half in range(2):
            accs = [jnp.zeros((16,), jnp.float32) for _ in range(16)]
            for l in range(_L):
                ids = idx_v[l, pl.ds(g * 16, 16)]
                w0 = ids * _DP + (half * 16)
                for c in range(16):
                    accs[c] = accs[c] + plsc.load_gather(tab_v, [w0 + c])
            for c in range(16):
                out_v[half * 16 + c, pl.ds(g * 16, 16)] = accs[c]
        return carry

    lax.fori_loop(0, _BAGS_PW // 16, bag_body, 0)
    pltpu.sync_copy(out_v, sout_hbm.at[wid])


_BB = 512


def _tc_body(dense_ref, small_ref, large_ref,
             bw0_ref, bb0_ref, bw1_ref, bb1_ref, bw2_ref, bb2_ref,
             U_ref, V_ref, db_ref,
             tw0_ref, tb0_ref, tw1_ref, tb1_ref, tw2_ref, tb2_ref,
             tw3_ref, tb3_ref, out_ref):
    f32 = jnp.float32
    h = dense_ref[...]
    h = jnp.maximum(jnp.dot(h, bw0_ref[...], preferred_element_type=f32)
                    + bb0_ref[...], 0.0)
    h = jnp.maximum(jnp.dot(h, bw1_ref[...], preferred_element_type=f32)
                    + bb1_ref[...], 0.0)
    h = jnp.maximum(jnp.dot(h, bw2_ref[...], preferred_element_type=f32)
                    + bb2_ref[...], 0.0)
    x0 = jnp.concatenate([h, small_ref[...], large_ref[...]], axis=1)
    xl = x0
    for i in range(3):
        t = jnp.dot(xl, U_ref[i], preferred_element_type=f32)
        t = jnp.dot(t, V_ref[i], preferred_element_type=f32) + db_ref[i]
        xl = x0 * t + xl
    h = jnp.maximum(jnp.dot(xl, tw0_ref[...], preferred_element_type=f32)
                    + tb0_ref[...], 0.0)
    h = jnp.maximum(jnp.dot(h, tw1_ref[...], preferred_element_type=f32)
                    + tb1_ref[...], 0.0)
    h = jnp.maximum(jnp.dot(h, tw2_ref[...], preferred_element_type=f32)
                    + tb2_ref[...], 0.0)
    out_ref[...] = jax.nn.sigmoid(
        jnp.dot(h, tw3_ref[...], preferred_element_type=f32) + tb3_ref[...])


def kernel(dense_input, large_emb_inputs, small_emb_inputs, large_tables,
           small_tables, bw0, bb0, bw1, bb1, bw2, bb2, dcn_U, dcn_V, dcn_b,
           tw0, tb0, tw1, tb1, tw2, tb2, tw3, tb3):
    f32 = jnp.float32
    si = small_emb_inputs.astype(jnp.int32)
    sidx = si.reshape(8, _BAGS_PW, _NS, _L).transpose(2, 0, 3, 1)
    sidx = sidx.reshape(_NW, _L, _BAGS_PW)
    stab = jnp.pad(small_tables.reshape(_NS * _VS, _D), ((0, 0), (0, 1)))
    sout = _small_bagsum(sidx, stab.reshape(_NS * _VS * _DP))
    small_cat = sout.reshape(_NS, 8, _D, _BAGS_PW).transpose(1, 3, 0, 2)
    small_cat = small_cat.reshape(_B, _NS * _D)

    liT = large_emb_inputs.astype(jnp.int32).T
    tabT = jnp.transpose(large_tables, (0, 2, 1)).reshape(_NROW, _VL)
    lout = _large_gather(liT, tabT)
    large_cat = lout.T

    row2 = lambda v: v.reshape(1, -1)
    full = lambda *shape: pl.BlockSpec(shape, lambda i: (0,) * len(shape))
    batched = lambda w: pl.BlockSpec((_BB, w), lambda i: (i, 0))

    out = pl.pallas_call(
        _tc_body,
        grid=(_B // _BB,),
        in_specs=[
            batched(_DIN), batched(_NS * _D), batched(_NL * _D),
            full(_DIN, 512), full(1, 512), full(512, 256), full(1, 256),
            full(256, 32), full(1, 32),
            full(3, 864, 128), full(3, 128, 864), full(3, 864),
            full(864, 1024), full(1, 1024), full(1024, 512), full(1, 512),
            full(512, 256), full(1, 256), full(256, 1), full(1, 1),
        ],
        out_specs=pl.BlockSpec((_BB, 1), lambda i: (i, 0)),
        out_shape=jax.ShapeDtypeStruct((_B, 1), f32),
    )(dense_input, small_cat, large_cat,
      bw0, row2(bb0), bw1, row2(bb1), bw2, row2(bb2),
      dcn_U, dcn_V, dcn_b,
      tw0, row2(tb0), tw1, row2(tb1), tw2, row2(tb2),
      tw3, row2(tb3))
    return out
